# baseline (speedup 1.0000x reference)
import functools
import math

import jax
import jax.numpy as jnp
from jax import lax
from jax.experimental import pallas as pl
from jax.experimental.pallas import tpu as pltpu
from jax.experimental.pallas import tpu_sc as plsc

B = 4096
L = 200
D = 64
SCALE = math.sqrt(D)

NW = 32
PER_W = B // NW
NBUF = 4
LANES = 16
RUNROLL = 8


def _sc_body(tok_hbm, table_hbm, out_hbm, idx_v, ins, outs, sem_g, sem_s):
    cid = lax.axis_index("c")
    sid = lax.axis_index("s")
    wid = sid * 2 + cid
    base = wid * PER_W

    pltpu.sync_copy(tok_hbm.at[pl.ds(base, PER_W)], idx_v)

    H0, H1 = 104, 96

    def start_gather(buf, bb):
        pltpu.make_async_copy(
            table_hbm.at[idx_v.at[bb, pl.ds(0, H0)]],
            ins[buf].at[pl.ds(0, H0)],
            sem_g.at[buf],
        ).start()
        pltpu.make_async_copy(
            table_hbm.at[idx_v.at[bb, pl.ds(H0, H1)]],
            ins[buf].at[pl.ds(H0, H1)],
            sem_g.at[buf],
        ).start()

    def wait_gather(buf, bb):
        pltpu.make_async_copy(
            table_hbm.at[idx_v.at[bb, pl.ds(0, H0)]],
            ins[buf].at[pl.ds(0, H0)],
            sem_g.at[buf],
        ).wait()
        pltpu.make_async_copy(
            table_hbm.at[idx_v.at[bb, pl.ds(H0, H1)]],
            ins[buf].at[pl.ds(H0, H1)],
            sem_g.at[buf],
        ).wait()

    def scale(buf):
        src, dst = ins[buf], outs[buf]

        def rowblk(i, carry):
            r0 = i * RUNROLL
            for rr in range(RUNROLL):
                for j in range(D // LANES):
                    sl = pl.ds(j * LANES, LANES)
                    dst[r0 + rr, sl] = src[r0 + rr, sl] * jnp.float32(SCALE)
            return carry

        lax.fori_loop(0, L // RUNROLL, rowblk, 0)

    for buf in range(NBUF):
        start_gather(buf, buf)

    def outer(t, carry):
        for buf in range(NBUF):
            bb = t * NBUF + buf
            wait_gather(buf, bb)
            @pl.when(bb >= NBUF)
            def _():
                pltpu.make_async_copy(
                    outs[buf], out_hbm.at[base + bb - NBUF], sem_s.at[buf]
                ).wait()

            scale(buf)

            @pl.when(bb + NBUF < PER_W)
            def _():
                start_gather(buf, bb + NBUF)

            pltpu.make_async_copy(
                outs[buf], out_hbm.at[base + bb], sem_s.at[buf]
            ).start()
        return carry

    lax.fori_loop(0, PER_W // NBUF, outer, 0)

    for buf in range(NBUF):
        bb = PER_W - NBUF + buf
        pltpu.make_async_copy(
            outs[buf], out_hbm.at[base + bb], sem_s.at[buf]
        ).wait()


_sc_gather = functools.partial(
    pl.kernel,
    mesh=plsc.VectorSubcoreMesh(core_axis_name="c", subcore_axis_name="s"),
    out_type=jax.ShapeDtypeStruct((B, L, D), jnp.float32),
    scratch_types=[
        pltpu.VMEM((PER_W, L), jnp.int32),
        [pltpu.VMEM((L, D), jnp.float32) for _ in range(NBUF)],
        [pltpu.VMEM((L, D), jnp.float32) for _ in range(NBUF)],
        pltpu.SemaphoreType.DMA((NBUF,)),
        pltpu.SemaphoreType.DMA((NBUF,)),
    ],
    compiler_params=pltpu.CompilerParams(use_tc_tiling_on_sc=False),
)(_sc_body)


def kernel(tokens, table):
    return _sc_gather(tokens, table)


# speedup vs baseline: 1.0017x; 1.0017x over previous
import functools
import math

import jax
import jax.numpy as jnp
from jax import lax
from jax.experimental import pallas as pl
from jax.experimental.pallas import tpu as pltpu
from jax.experimental.pallas import tpu_sc as plsc

B = 4096
L = 200
D = 64
SCALE = math.sqrt(D)

NW = 32
PER_W = B // NW
NBUF = 4
LANES = 16
RUNROLL = 8


def _sc_body(tok_hbm, table_hbm, out_hbm, idx_v, ins, outs, sem_g, sem_s):
    cid = lax.axis_index("c")
    sid = lax.axis_index("s")
    wid = sid * 2 + cid
    base = wid * PER_W

    pltpu.sync_copy(tok_hbm.at[pl.ds(base, PER_W), pl.ds(0, L)], idx_v)

    H0, H1 = 104, 96

    def start_gather(buf, bb):
        pltpu.make_async_copy(
            table_hbm.at[idx_v.at[bb, pl.ds(0, H0)]],
            ins[buf].at[pl.ds(0, H0)],
            sem_g.at[buf],
        ).start()
        pltpu.make_async_copy(
            table_hbm.at[idx_v.at[bb, pl.ds(H0, H1)]],
            ins[buf].at[pl.ds(H0, H1)],
            sem_g.at[buf],
        ).start()

    def wait_gather(buf, bb):
        pltpu.make_async_copy(
            table_hbm.at[idx_v.at[bb, pl.ds(0, H0)]],
            ins[buf].at[pl.ds(0, H0)],
            sem_g.at[buf],
        ).wait()
        pltpu.make_async_copy(
            table_hbm.at[idx_v.at[bb, pl.ds(H0, H1)]],
            ins[buf].at[pl.ds(H0, H1)],
            sem_g.at[buf],
        ).wait()

    def scale(buf):
        src, dst = ins[buf], outs[buf]

        def rowblk(i, carry):
            r0 = i * RUNROLL
            for rr in range(RUNROLL):
                for j in range(D // LANES):
                    sl = pl.ds(j * LANES, LANES)
                    dst[r0 + rr, sl] = src[r0 + rr, sl] * jnp.float32(SCALE)
            return carry

        lax.fori_loop(0, L // RUNROLL, rowblk, 0)

    for buf in range(NBUF):
        start_gather(buf, buf)

    def outer(t, carry):
        for buf in range(NBUF):
            bb = t * NBUF + buf
            wait_gather(buf, bb)
            @pl.when(bb >= NBUF)
            def _():
                pltpu.make_async_copy(
                    outs[buf], out_hbm.at[base + bb - NBUF], sem_s.at[buf]
                ).wait()

            scale(buf)

            @pl.when(bb + NBUF < PER_W)
            def _():
                start_gather(buf, bb + NBUF)

            pltpu.make_async_copy(
                outs[buf], out_hbm.at[base + bb], sem_s.at[buf]
            ).start()
        return carry

    lax.fori_loop(0, PER_W // NBUF, outer, 0)

    for buf in range(NBUF):
        bb = PER_W - NBUF + buf
        pltpu.make_async_copy(
            outs[buf], out_hbm.at[base + bb], sem_s.at[buf]
        ).wait()


_sc_gather = functools.partial(
    pl.kernel,
    mesh=plsc.VectorSubcoreMesh(core_axis_name="c", subcore_axis_name="s"),
    out_type=jax.ShapeDtypeStruct((B, L, D), jnp.float32),
    scratch_types=[
        pltpu.VMEM((PER_W, L), jnp.int32),
        [pltpu.VMEM((L, D), jnp.float32) for _ in range(NBUF)],
        [pltpu.VMEM((L, D), jnp.float32) for _ in range(NBUF)],
        pltpu.SemaphoreType.DMA((NBUF,)),
        pltpu.SemaphoreType.DMA((NBUF,)),
    ],
    compiler_params=pltpu.CompilerParams(use_tc_tiling_on_sc=False),
)(_sc_body)


def kernel(tokens, table):
    tok = jnp.pad(tokens, ((0, 0), (0, 256 - L)))
    return _sc_gather(tok, table)


# speedup vs baseline: 1.0045x; 1.0028x over previous
import functools
import math

import jax
import jax.numpy as jnp
from jax import lax
from jax.experimental import pallas as pl
from jax.experimental.pallas import tpu as pltpu
from jax.experimental.pallas import tpu_sc as plsc

B = 4096
L = 200
D = 64
SCALE = math.sqrt(D)

NW = 32
ROWS = B * L
PER_W = ROWS // NW
BROWS_W = B // NW
C = 128
G = PER_W // C
NBUF = 4
LANES = 16
RUNROLL = 8


def _depad_body(tok_hbm, flat_hbm, tok_v, flat_v):
    cid = lax.axis_index("c")
    sid = lax.axis_index("s")
    wid = sid * 2 + cid
    base = wid * BROWS_W

    pltpu.sync_copy(tok_hbm.at[pl.ds(base, BROWS_W)], tok_v)

    offs = [16 * k for k in range(12)] + [184]

    def row(r, carry):
        for o in offs:
            flat_v[pl.ds(r * L + o, LANES)] = tok_v[r, pl.ds(o, LANES)]
        return carry

    lax.fori_loop(0, BROWS_W, row, 0)
    pltpu.sync_copy(flat_v, flat_hbm.at[pl.ds(wid * PER_W, PER_W)])


_depad = functools.partial(
    pl.kernel,
    mesh=plsc.VectorSubcoreMesh(core_axis_name="c", subcore_axis_name="s"),
    out_type=jax.ShapeDtypeStruct((ROWS,), jnp.int32),
    scratch_types=[
        pltpu.VMEM((BROWS_W, L), jnp.int32),
        pltpu.VMEM((PER_W,), jnp.int32),
    ],
)(_depad_body)


def _gather_body(tok_hbm, table_hbm, out_hbm, idx_v, ins, outs, sem_g, sem_s):
    cid = lax.axis_index("c")
    sid = lax.axis_index("s")
    wid = sid * 2 + cid
    base = wid * PER_W

    pltpu.sync_copy(tok_hbm.at[pl.ds(base, PER_W)], idx_v)

    def start_gather(buf, g):
        pltpu.make_async_copy(
            table_hbm.at[idx_v.at[pl.ds(g * C, C)]], ins[buf], sem_g.at[buf]
        ).start()

    def wait_gather(buf, g):
        pltpu.make_async_copy(
            table_hbm.at[idx_v.at[pl.ds(g * C, C)]], ins[buf], sem_g.at[buf]
        ).wait()

    def scale(buf):
        src, dst = ins[buf], outs[buf]

        def rowblk(i, carry):
            r0 = i * RUNROLL
            for rr in range(RUNROLL):
                for j in range(D // LANES):
                    sl = pl.ds(j * LANES, LANES)
                    dst[r0 + rr, sl] = src[r0 + rr, sl] * jnp.float32(SCALE)
            return carry

        lax.fori_loop(0, C // RUNROLL, rowblk, 0)

    for buf in range(NBUF):
        start_gather(buf, buf)

    def outer(t, carry):
        for buf in range(NBUF):
            g = t * NBUF + buf
            wait_gather(buf, g)
            @pl.when(g >= NBUF)
            def _():
                pltpu.make_async_copy(
                    outs[buf],
                    out_hbm.at[pl.ds(base + (g - NBUF) * C, C)],
                    sem_s.at[buf],
                ).wait()

            scale(buf)

            @pl.when(g + NBUF < G)
            def _():
                start_gather(buf, g + NBUF)

            pltpu.make_async_copy(
                outs[buf], out_hbm.at[pl.ds(base + g * C, C)], sem_s.at[buf]
            ).start()
        return carry

    lax.fori_loop(0, G // NBUF, outer, 0)

    for buf in range(NBUF):
        g = G - NBUF + buf
        pltpu.make_async_copy(
            outs[buf], out_hbm.at[pl.ds(base + g * C, C)], sem_s.at[buf]
        ).wait()


_sc_gather = functools.partial(
    pl.kernel,
    mesh=plsc.VectorSubcoreMesh(core_axis_name="c", subcore_axis_name="s"),
    out_type=jax.ShapeDtypeStruct((ROWS, D), jnp.float32),
    scratch_types=[
        pltpu.VMEM((PER_W,), jnp.int32),
        [pltpu.VMEM((C, D), jnp.float32) for _ in range(NBUF)],
        [pltpu.VMEM((C, D), jnp.float32) for _ in range(NBUF)],
        pltpu.SemaphoreType.DMA((NBUF,)),
        pltpu.SemaphoreType.DMA((NBUF,)),
    ],
    compiler_params=pltpu.CompilerParams(use_tc_tiling_on_sc=False),
)(_gather_body)


def kernel(tokens, table):
    flat = _depad(tokens)
    out = _sc_gather(flat, table)
    return out.reshape(B, L, D)


# speedup vs baseline: 1.2158x; 1.2104x over previous
import functools
import math

import jax
import jax.numpy as jnp
from jax import lax
from jax.experimental import pallas as pl
from jax.experimental.pallas import tpu as pltpu
from jax.experimental.pallas import tpu_sc as plsc

B = 4096
L = 200
D = 64
DP = 128
SCALE = math.sqrt(D)

NW = 32
ROWS = B * L
PER_W = ROWS // NW
BROWS_W = B // NW
TBLK = 32
C = 80
G = PER_W // C
NBUF = 4
LANES = 16
RUNROLL = 8


def _sc_body(tok_hbm, table_hbm, out_hbm, tok_v, idx_v, ins, outs,
             sem_g, sem_s):
    cid = lax.axis_index("c")
    sid = lax.axis_index("s")
    wid = sid * 2 + cid
    bbase = wid * BROWS_W
    base = wid * PER_W

    offs = [16 * k for k in range(12)] + [184]

    for blk in range(BROWS_W // TBLK):
        pltpu.sync_copy(tok_hbm.at[pl.ds(bbase + blk * TBLK, TBLK)], tok_v)

        def row(r, carry, blk=blk):
            for o in offs:
                idx_v[pl.ds((blk * TBLK + r) * L + o, LANES)] = (
                    tok_v[r, pl.ds(o, LANES)]
                )
            return carry

        lax.fori_loop(0, TBLK, row, 0)

    def start_gather(buf, g):
        pltpu.make_async_copy(
            table_hbm.at[idx_v.at[pl.ds(g * C, C)]], ins[buf], sem_g.at[buf]
        ).start()

    def wait_gather(buf, g):
        pltpu.make_async_copy(
            table_hbm.at[idx_v.at[pl.ds(g * C, C)]], ins[buf], sem_g.at[buf]
        ).wait()

    def scale(buf):
        src, dst = ins[buf], outs[buf]

        def rowblk(i, carry):
            r0 = i * RUNROLL
            for rr in range(RUNROLL):
                for j in range(D // LANES):
                    sl = pl.ds(j * LANES, LANES)
                    dst[r0 + rr, sl] = src[r0 + rr, sl] * jnp.float32(SCALE)
            return carry

        lax.fori_loop(0, C // RUNROLL, rowblk, 0)

    for buf in range(NBUF):
        start_gather(buf, buf)

    def outer(t, carry):
        for buf in range(NBUF):
            g = t * NBUF + buf
            wait_gather(buf, g)
            @pl.when(g >= NBUF)
            def _():
                pltpu.make_async_copy(
                    outs[buf],
                    out_hbm.at[pl.ds(base + (g - NBUF) * C, C)],
                    sem_s.at[buf],
                ).wait()

            scale(buf)

            @pl.when(g + NBUF < G)
            def _():
                start_gather(buf, g + NBUF)

            pltpu.make_async_copy(
                outs[buf], out_hbm.at[pl.ds(base + g * C, C)], sem_s.at[buf]
            ).start()
        return carry

    lax.fori_loop(0, G // NBUF, outer, 0)

    for buf in range(NBUF):
        g = G - NBUF + buf
        pltpu.make_async_copy(
            outs[buf], out_hbm.at[pl.ds(base + g * C, C)], sem_s.at[buf]
        ).wait()


_sc_gather = functools.partial(
    pl.kernel,
    mesh=plsc.VectorSubcoreMesh(core_axis_name="c", subcore_axis_name="s"),
    out_type=jax.ShapeDtypeStruct((ROWS, D), jnp.float32),
    scratch_types=[
        pltpu.VMEM((TBLK, L), jnp.int32),
        pltpu.VMEM((PER_W,), jnp.int32),
        [pltpu.VMEM((C, DP), jnp.float32) for _ in range(NBUF)],
        [pltpu.VMEM((C, D), jnp.float32) for _ in range(NBUF)],
        pltpu.SemaphoreType.DMA((NBUF,)),
        pltpu.SemaphoreType.DMA((NBUF,)),
    ],
)(_sc_body)


def kernel(tokens, table):
    tpad = jnp.pad(table, ((0, 0), (0, DP - D)))
    out = _sc_gather(tokens, tpad)
    return out.reshape(B, L, D)
